# baseline (speedup 1.0000x reference)
import functools

import jax
import jax.numpy as jnp
from jax import lax
from jax.experimental import pallas as pl
from jax.experimental.pallas import tpu as pltpu
from jax.experimental.pallas import tpu_sc as plsc

DIM = 64
NC = 2
NS = 16
NW = NC * NS
BLK = 128
NGBUF = 4
NOBUF = 2


def _make_repack(V):
    n_cols = (V + BLK - 1) // BLK
    mesh = plsc.VectorSubcoreMesh(core_axis_name="c", subcore_axis_name="s")
    tail_w = (n_cols - 1) % NW

    @functools.partial(
        pl.kernel,
        mesh=mesh,
        out_type=jax.ShapeDtypeStruct((V // 2, 2 * DIM), jnp.float32),
        compiler_params=pltpu.CompilerParams(
            use_tc_tiling_on_sc=True, needs_layout_passes=False
        ),
        scratch_types=[
            pltpu.VMEM((4, DIM, BLK), jnp.float32),
            pltpu.VMEM((2, DIM, BLK + 1), jnp.float32),
            pltpu.SemaphoreType.DMA,
            pltpu.SemaphoreType.DMA,
            pltpu.SemaphoreType.DMA,
            pltpu.SemaphoreType.DMA,
            pltpu.SemaphoreType.DMA,
            pltpu.SemaphoreType.DMA,
            pltpu.SemaphoreType.DMA,
        ],
    )
    def repack_kernel(wt_hbm, out_hbm, slab_v, stage_v,
                      r0, r1, r2, r3, o0, o1, ot):
        rsem = (r0, r1, r2, r3)
        osem = (o0, o1)
        wid = lax.axis_index("s") * NC + lax.axis_index("c")
        n_iter = (n_cols + NW - 1) // NW

        def fire_read(t, p):
            col = wid + NW * t
            pltpu.async_copy(
                wt_hbm.at[:, pl.ds(col * BLK, BLK)], slab_v.at[p], rsem[p]
            )

        for p0 in range(4):
            fire_read(p0, p0)

        lane = lax.iota(jnp.int32, 16)

        def body(g, carry):
            for h in range(4):
                t = g * 4 + h
                p = h
                q = h % 2
                col = wid + NW * t

                @pl.when(col < n_cols)
                def _work():
                    pltpu.make_async_copy(
                        wt_hbm.at[:, pl.ds(col * BLK, BLK)], slab_v.at[p],
                        rsem[p],
                    ).wait()

                    @pl.when(t >= 2)
                    def _drain_out():
                        pltpu.make_async_copy(
                            stage_v.at[q, :, pl.ds(0, BLK)],
                            out_hbm.at[pl.ds(col * (BLK // 2), BLK // 2)],
                            osem[q],
                        ).wait()

                    @plsc.parallel_loop(0, DIM, unroll=8)
                    def _row(d):
                        dv = jnp.full((16,), d, jnp.int32)
                        for ib in range(BLK // 16):
                            vec = slab_v[p, d, pl.ds(16 * ib, 16)]
                            vcol = lane + 16 * ib
                            plsc.store_scatter(
                                stage_v.at[q],
                                [vcol >> 1, (vcol & 1) * DIM + dv],
                                vec,
                            )

                    @pl.when(col < n_cols - 1)
                    def _full():
                        pltpu.async_copy(
                            stage_v.at[q, :, pl.ds(0, BLK)],
                            out_hbm.at[pl.ds(col * (BLK // 2), BLK // 2)],
                            osem[q],
                        )

                    @pl.when(col == n_cols - 1)
                    def _tail():
                        pltpu.async_copy(
                            stage_v.at[q, pl.ds(0, DIM // 2), pl.ds(0, BLK)],
                            out_hbm.at[pl.ds(col * (BLK // 2), BLK // 4)],
                            ot,
                        )

                    @pl.when(col + 4 * NW < n_cols)
                    def _next():
                        fire_read(t + 4, p)

            return carry

        lax.fori_loop(0, (n_iter + 3) // 4, body, 0)

        @pl.when(wid != tail_w)
        def _drain0():
            pltpu.make_async_copy(
                stage_v.at[0, :, pl.ds(0, BLK)],
                out_hbm.at[pl.ds(0, BLK // 2)], osem[0],
            ).wait()

        pltpu.make_async_copy(
            stage_v.at[1, :, pl.ds(0, BLK)],
            out_hbm.at[pl.ds(0, BLK // 2)], osem[1],
        ).wait()

        @pl.when(wid == tail_w)
        def _drain_tail():
            pltpu.make_async_copy(
                stage_v.at[0, pl.ds(0, DIM // 2), pl.ds(0, BLK)],
                out_hbm.at[pl.ds(0, BLK // 4)], ot,
            ).wait()


    return repack_kernel


def _make_gather(L, B):
    assert B % (NW * BLK) == 0 and B // NW == BLK
    assert L % NGBUF == 0
    mesh = plsc.VectorSubcoreMesh(core_axis_name="c", subcore_axis_name="s")

    @functools.partial(
        pl.kernel,
        mesh=mesh,
        out_type=jax.ShapeDtypeStruct((L, DIM, B), jnp.float32),
        compiler_params=pltpu.CompilerParams(
            use_tc_tiling_on_sc=False, needs_layout_passes=False
        ),
        scratch_types=[
            pltpu.VMEM((L, BLK), jnp.int32),
            pltpu.VMEM((NGBUF, BLK, DIM), jnp.float32),
            pltpu.VMEM((NOBUF, DIM, BLK + 1), jnp.float32),
            pltpu.SemaphoreType.DMA,
            pltpu.SemaphoreType.DMA,
            pltpu.SemaphoreType.DMA,
            pltpu.SemaphoreType.DMA,
            pltpu.SemaphoreType.DMA,
            pltpu.SemaphoreType.DMA,
        ],
    )
    def gather_kernel(idx_hbm, table_hbm, out_hbm, idx_v, rows_v, stage_v,
                      g0, g1, g2, g3, o0, o1):
        gsem = (g0, g1, g2, g3)
        osem = (o0, o1)
        wid = lax.axis_index("s") * NC + lax.axis_index("c")
        b0 = wid * BLK

        pltpu.sync_copy(idx_hbm.at[:, pl.ds(b0, BLK)], idx_v)

        def fire_gather(l, p):
            return pltpu.async_copy(
                table_hbm.at[idx_v.at[l]], rows_v.at[p], gsem[p]
            )

        for p in range(NGBUF):
            fire_gather(p, p)

        lane = lax.iota(jnp.int32, 16)
        rids = [lane + (16 * ib) for ib in range(DIM // 16)]

        def body(g, carry):
            for l in range(NGBUF):
                labs = g * NGBUF + l
                p = l
                q = l % NOBUF

                pltpu.make_async_copy(
                    table_hbm.at[idx_v.at[labs]], rows_v.at[p], gsem[p]
                ).wait()

                @pl.when(jnp.logical_or(l >= NOBUF, g > 0))
                def _drain_out():
                    pltpu.make_async_copy(
                        stage_v.at[q, :, pl.ds(0, BLK)],
                        out_hbm.at[labs, :, pl.ds(b0, BLK)], osem[q],
                    ).wait()

                @plsc.parallel_loop(0, BLK, unroll=8)
                def _row(i):
                    iv = jnp.full((16,), i, jnp.int32)
                    for j in range(DIM // 16):
                        vec = rows_v[p, i, pl.ds(16 * j, 16)]
                        plsc.store_scatter(stage_v.at[q], [rids[j], iv], vec)

                pltpu.async_copy(
                    stage_v.at[q, :, pl.ds(0, BLK)],
                    out_hbm.at[labs, :, pl.ds(b0, BLK)], osem[q]
                )

                @pl.when(labs + NGBUF < L)
                def _refill():
                    fire_gather(labs + NGBUF, p)
            return carry

        lax.fori_loop(0, L // NGBUF, body, 0)
        for q in range(NOBUF):
            pltpu.make_async_copy(
                stage_v.at[q, :, pl.ds(0, BLK)],
                out_hbm.at[0, :, pl.ds(b0, BLK)], osem[q]
            ).wait()

    return gather_kernel


def kernel(input, weight):
    B, L = input.shape
    V = weight.shape[0]
    idx_t = input.T.astype(jnp.int32)
    pairs = _make_repack(V)(weight.T)
    table = pairs.reshape(V, DIM)
    out_t = _make_gather(L, B)(idx_t, table)
    return jnp.transpose(out_t, (2, 0, 1))


# speedup vs baseline: 1.2799x; 1.2799x over previous
import functools

import jax
import jax.numpy as jnp
from jax import lax
from jax.experimental import pallas as pl
from jax.experimental.pallas import tpu as pltpu
from jax.experimental.pallas import tpu_sc as plsc

DIM = 64
NC = 2
NS = 16
NW = NC * NS
BLK = 128
NGBUF = 4
NOBUF = 2


def _make_gather(L, B):
    assert B % (NW * BLK) == 0 and B // NW == BLK
    assert L % NGBUF == 0
    mesh = plsc.VectorSubcoreMesh(core_axis_name="c", subcore_axis_name="s")

    @functools.partial(
        pl.kernel,
        mesh=mesh,
        out_type=jax.ShapeDtypeStruct((L, DIM, B), jnp.float32),
        compiler_params=pltpu.CompilerParams(
            use_tc_tiling_on_sc=False, needs_layout_passes=False
        ),
        scratch_types=[
            pltpu.VMEM((L, BLK), jnp.int32),
            pltpu.VMEM((NGBUF, BLK, DIM), jnp.float32),
            pltpu.VMEM((NOBUF, DIM, BLK + 1), jnp.float32),
            pltpu.SemaphoreType.DMA,
            pltpu.SemaphoreType.DMA,
            pltpu.SemaphoreType.DMA,
            pltpu.SemaphoreType.DMA,
            pltpu.SemaphoreType.DMA,
            pltpu.SemaphoreType.DMA,
        ],
    )
    def gather_kernel(idx_hbm, table_hbm, out_hbm, idx_v, rows_v, stage_v,
                      g0, g1, g2, g3, o0, o1):
        gsem = (g0, g1, g2, g3)
        osem = (o0, o1)
        wid = lax.axis_index("s") * NC + lax.axis_index("c")
        b0 = wid * BLK

        pltpu.sync_copy(idx_hbm.at[:, pl.ds(b0, BLK)], idx_v)

        def fire_gather(l, p):
            return pltpu.async_copy(
                table_hbm.at[idx_v.at[l]], rows_v.at[p], gsem[p]
            )

        for p in range(NGBUF):
            fire_gather(p, p)

        lane = lax.iota(jnp.int32, 16)
        rids = [lane + (16 * ib) for ib in range(BLK // 16)]

        def body(g, carry):
            for l in range(NGBUF):
                labs = g * NGBUF + l
                p = l
                q = l % NOBUF

                pltpu.make_async_copy(
                    table_hbm.at[idx_v.at[labs]], rows_v.at[p], gsem[p]
                ).wait()

                @pl.when(jnp.logical_or(l >= NOBUF, g > 0))
                def _drain_out():
                    pltpu.make_async_copy(
                        stage_v.at[q, :, pl.ds(0, BLK)],
                        out_hbm.at[labs, :, pl.ds(b0, BLK)], osem[q],
                    ).wait()

                @plsc.parallel_loop(0, BLK, unroll=8)
                def _row(i):
                    iv = jnp.full((16,), i, jnp.int32)
                    for j in range(DIM // 16):
                        vec = rows_v[p, i, pl.ds(16 * j, 16)]
                        plsc.store_scatter(stage_v.at[q], [rids[j], iv], vec)

                pltpu.async_copy(
                    stage_v.at[q, :, pl.ds(0, BLK)],
                    out_hbm.at[labs, :, pl.ds(b0, BLK)], osem[q]
                )

                @pl.when(labs + NGBUF < L)
                def _refill():
                    fire_gather(labs + NGBUF, p)
            return carry

        lax.fori_loop(0, L // NGBUF, body, 0)
        for q in range(NOBUF):
            pltpu.make_async_copy(
                stage_v.at[q, :, pl.ds(0, BLK)],
                out_hbm.at[0, :, pl.ds(b0, BLK)], osem[q]
            ).wait()

    return gather_kernel


def kernel(input, weight):
    B, L = input.shape
    idx_t = input.T.astype(jnp.int32)
    out_t = _make_gather(L, B)(idx_t, weight)
    return jnp.transpose(out_t, (2, 0, 1))
